# Initial kernel scaffold; baseline (speedup 1.0000x reference)
import jax
import jax.numpy as jnp
from jax.experimental import pallas as pl


def kernel(node_features, edge_index, edge_features, W_self, b_self, W_msg, b_msg, ln_gamma, ln_beta):
    raise NotImplementedError("write your pallas kernel here")


# speedup vs baseline: 2.3703x; 2.3703x over previous
import functools

import jax
import jax.numpy as jnp
from jax import lax
from jax.experimental import pallas as pl
from jax.experimental.pallas import tpu as pltpu
from jax.experimental.pallas import tpu_sc as plsc

N = 10000
E = 160000
D = 256
DE = 16
DO = 256
DH = D // 2
EW = 16

NUM_CORES = 2
NUM_SUBCORES = 16
EPT = E // NUM_SUBCORES
C = 80
ITERS = EPT // C
NP = 10240
RPT = NP // NUM_SUBCORES
ZR = 128


def _zero_fill(ref):
    rows, width = ref.shape
    nv = width // 16

    def body(i, carry):
        ref[i // nv, pl.ds((i % nv) * 16, 16)] = jnp.zeros((16,), jnp.float32)
        return carry

    lax.fori_loop(0, rows * nv, body, 0)


def _sc_body(xlo_hbm, xhi_hbm, e0_hbm, e1_hbm, src_hbm, dst_hbm,
             alo_out, ahi_out, e0_out, e1_out,
             src_v, dst_v, xrows, erows, zA, zE, A_sh, E_sh, sem):
    c = lax.axis_index("c")
    s = lax.axis_index("s")

    _zero_fill(zA)
    _zero_fill(zE)
    for j in range(RPT // ZR):
        lo = s * RPT + j * ZR
        pltpu.sync_copy(zA, A_sh.at[pl.ds(lo, ZR)])
        pltpu.sync_copy(zE, E_sh.at[pl.ds(lo, ZR)])
    plsc.subcore_barrier()

    def make_loop(x_hbm, e_hbm):
        def body(k, carry):
            base = s * EPT + k * C
            pltpu.sync_copy(src_hbm.at[pl.ds(base, C)], src_v)
            pltpu.sync_copy(dst_hbm.at[pl.ds(base, C)], dst_v)
            pltpu.async_copy(x_hbm.at[src_v], xrows, sem).wait()
            pltpu.sync_copy(e_hbm.at[pl.ds(base, C)], erows)
            pltpu.sync_copy(xrows, A_sh.at[dst_v], add=True)
            pltpu.sync_copy(erows, E_sh.at[dst_v], add=True)
            return carry
        lax.fori_loop(0, ITERS, body, 0)

    @pl.when(c == 0)
    def _():
        make_loop(xlo_hbm, e0_hbm)

    @pl.when(c == 1)
    def _():
        make_loop(xhi_hbm, e1_hbm)

    plsc.subcore_barrier()

    lo = s * RPT

    @pl.when(c == 0)
    def _():
        pltpu.sync_copy(A_sh.at[pl.ds(lo, RPT)], alo_out.at[pl.ds(lo, RPT)])
        pltpu.sync_copy(E_sh.at[pl.ds(lo, RPT)], e0_out.at[pl.ds(lo, RPT)])

    @pl.when(c == 1)
    def _():
        pltpu.sync_copy(A_sh.at[pl.ds(lo, RPT)], ahi_out.at[pl.ds(lo, RPT)])
        pltpu.sync_copy(E_sh.at[pl.ds(lo, RPT)], e1_out.at[pl.ds(lo, RPT)])


def _sc_aggregate(x_lo, x_hi, e0, e1, src, dst):
    mesh = plsc.VectorSubcoreMesh(core_axis_name="c", subcore_axis_name="s")
    f32 = jnp.float32
    run = pl.kernel(
        _sc_body,
        mesh=mesh,
        compiler_params=pltpu.CompilerParams(use_tc_tiling_on_sc=False),
        out_type=[
            jax.ShapeDtypeStruct((NP, DH), f32),
            jax.ShapeDtypeStruct((NP, DH), f32),
            jax.ShapeDtypeStruct((NP, EW), f32),
            jax.ShapeDtypeStruct((NP, EW), f32),
        ],
        scratch_types=[
            pltpu.VMEM((C,), jnp.int32),
            pltpu.VMEM((C,), jnp.int32),
            pltpu.VMEM((C, DH), f32),
            pltpu.VMEM((C, EW), f32),
            pltpu.VMEM((ZR, DH), f32),
            pltpu.VMEM((ZR, EW), f32),
            pltpu.VMEM_SHARED((NP, DH), f32),
            pltpu.VMEM_SHARED((NP, EW), f32),
            pltpu.SemaphoreType.DMA,
        ],
    )
    return run(x_lo, x_hi, e0, e1, src, dst)


BN = 1000


def _tc_body(x_ref, alo_ref, ahi_ref, e0_ref, e1_ref,
             ws_ref, wlo_ref, whi_ref, w0_ref, w1_ref,
             bs_ref, g_ref, b_ref, o_ref):
    hi = jax.lax.Precision.HIGHEST
    agg = jnp.dot(alo_ref[...], wlo_ref[...], precision=hi,
                  preferred_element_type=jnp.float32)
    agg = agg + jnp.dot(ahi_ref[...], whi_ref[...], precision=hi,
                        preferred_element_type=jnp.float32)
    agg = agg + jnp.dot(e0_ref[...], w0_ref[...], precision=hi,
                        preferred_element_type=jnp.float32)
    agg = agg + jnp.dot(e1_ref[...], w1_ref[...], precision=hi,
                        preferred_element_type=jnp.float32)
    deg = e0_ref[...][:, 8:9]
    agg = agg / jnp.maximum(deg, 1.0)
    comb = jnp.dot(x_ref[...], ws_ref[...], precision=hi,
                   preferred_element_type=jnp.float32)
    comb = comb + bs_ref[...] + agg
    mean = jnp.mean(comb, axis=1, keepdims=True)
    cen = comb - mean
    var = jnp.mean(cen * cen, axis=1, keepdims=True)
    normed = cen * lax.rsqrt(var + 1e-5)
    out = normed * g_ref[...] + b_ref[...]
    o_ref[...] = jnp.maximum(out, 0.0)


def _tc_combine(x, a_lo, a_hi, e0a, e1a, W_self, Wm_lo, Wm_hi, Wm0, Wm1,
                b_self, ln_gamma, ln_beta):
    grid = (N // BN,)
    full = lambda i: (0, 0)
    row = lambda i: (i, 0)
    return pl.pallas_call(
        _tc_body,
        grid=grid,
        in_specs=[
            pl.BlockSpec((BN, D), row),
            pl.BlockSpec((BN, DH), row),
            pl.BlockSpec((BN, DH), row),
            pl.BlockSpec((BN, EW), row),
            pl.BlockSpec((BN, EW), row),
            pl.BlockSpec((D, DO), full),
            pl.BlockSpec((DH, DO), full),
            pl.BlockSpec((DH, DO), full),
            pl.BlockSpec((EW, DO), full),
            pl.BlockSpec((EW, DO), full),
            pl.BlockSpec((1, DO), full),
            pl.BlockSpec((1, DO), full),
            pl.BlockSpec((1, DO), full),
        ],
        out_specs=pl.BlockSpec((BN, DO), row),
        out_shape=jax.ShapeDtypeStruct((N, DO), jnp.float32),
    )(x, a_lo, a_hi, e0a, e1a, W_self, Wm_lo, Wm_hi, Wm0, Wm1,
      b_self, ln_gamma, ln_beta)


@jax.jit
def _run(node_features, edge_index, edge_features, W_self, b_self,
         W_msg, b_msg, ln_gamma, ln_beta):
    f32 = jnp.float32
    src = edge_index[0]
    dst = edge_index[1]
    x_lo = node_features[:, :DH]
    x_hi = node_features[:, DH:]
    e0 = jnp.concatenate(
        [edge_features[:, :8], jnp.ones((E, 1), f32), jnp.zeros((E, 7), f32)],
        axis=1)
    e1 = jnp.concatenate([edge_features[:, 8:], jnp.zeros((E, 8), f32)], axis=1)

    a_lo, a_hi, e0a, e1a = _sc_aggregate(x_lo, x_hi, e0, e1, src, dst)

    Wm_lo = W_msg[:DH]
    Wm_hi = W_msg[DH:D]
    Wm0 = jnp.concatenate(
        [W_msg[D:D + 8], b_msg[None, :], jnp.zeros((7, DO), f32)], axis=0)
    Wm1 = jnp.concatenate([W_msg[D + 8:D + 16], jnp.zeros((8, DO), f32)], axis=0)

    return _tc_combine(node_features, a_lo, a_hi, e0a, e1a,
                       W_self, Wm_lo, Wm_hi, Wm0, Wm1,
                       b_self[None, :], ln_gamma[None, :], ln_beta[None, :])


def kernel(node_features, edge_index, edge_features, W_self, b_self,
           W_msg, b_msg, ln_gamma, ln_beta):
    return _run(node_features, edge_index, edge_features, W_self, b_self,
                W_msg, b_msg, ln_gamma, ln_beta)
